# baseline (speedup 1.0000x reference)
import jax
import jax.numpy as jnp
from jax.experimental import pallas as pl
from jax.experimental.pallas import tpu as pltpu

_N_DMA_THREADS = 2


def _pos_kernel(row_ref, col_ref, out_ref, scratch, sems):
    b, two_d, hw = out_ref.shape
    d = two_d // 2
    h = row_ref.shape[0]
    w = col_ref.shape[0]
    ceT = jnp.transpose(col_ref[:, :])
    reT = jnp.transpose(row_ref[:, :])
    scratch[:d] = jnp.broadcast_to(ceT[:, None, :], (d, h, w)).reshape(d, hw)
    scratch[d:] = jnp.broadcast_to(reT[:, :, None], (d, h, w)).reshape(d, hw)
    for i in range(b):
        pltpu.make_async_copy(scratch, out_ref.at[i], sems.at[i]).start(
            priority=i % _N_DMA_THREADS)
    for i in range(b):
        pltpu.make_async_copy(scratch, out_ref.at[i], sems.at[i]).wait()


def kernel(x, row_embed, col_embed):
    b = x.shape[0]
    h, w = x.shape[-2], x.shape[-1]
    d = row_embed.shape[-1]
    out = pl.pallas_call(
        _pos_kernel,
        in_specs=[
            pl.BlockSpec((h, d), lambda: (0, 0)),
            pl.BlockSpec((w, d), lambda: (0, 0)),
        ],
        out_specs=pl.BlockSpec(memory_space=pl.ANY),
        out_shape=jax.ShapeDtypeStruct((b, 2 * d, h * w), row_embed.dtype),
        scratch_shapes=[
            pltpu.VMEM((2 * d, h * w), row_embed.dtype),
            pltpu.SemaphoreType.DMA((b,)),
        ],
    )(row_embed[:h], col_embed[:w])
    return out.reshape(b, 2 * d, h, w)
